# Initial kernel scaffold; baseline (speedup 1.0000x reference)
import jax
import jax.numpy as jnp
from jax.experimental import pallas as pl


def kernel(y_pr, y_gt):
    raise NotImplementedError("write your pallas kernel here")


# speedup vs baseline: 7.3072x; 7.3072x over previous
import jax
import jax.numpy as jnp
from jax import lax
from jax.experimental import pallas as pl
from jax.experimental.pallas import tpu as pltpu

_NEG_RATIO = 3.0
_BAL_SCALE = 5.0
_EPS = 1e-09
_L1_SCALE = 10.0
_DICE_EPS = 0.001

_B = 8
_ROWS_PER_B = 400
_LANES = 1024
_RB = 80
_JB = _ROWS_PER_B // _RB
_ROWS = _B * _ROWS_PER_B
_BISECT_ITERS = 30
_CH = 40
_NCH = _ROWS // _CH

_BIG = 3.0e38


def _body(pr_ref, gt_ref, out_ref, neg_ref, sum_ref, mm_ref, dice_ref):
    b = pl.program_id(0)
    j = pl.program_id(1)

    @pl.when(jnp.logical_and(b == 0, j == 0))
    def _init():
        sum_ref[...] = jnp.zeros((8, _LANES), jnp.float32)
        dice_ref[...] = jnp.zeros((8, _B, _LANES), jnp.float32)
        mm_ref[...] = jnp.zeros((8, _LANES), jnp.float32)
        mm_ref[0:1, :] = jnp.full((1, _LANES), _BIG, jnp.float32)
        mm_ref[1:2, :] = jnp.full((1, _LANES), -_BIG, jnp.float32)

    binary = pr_ref[0, 0]
    thresh_binary = pr_ref[0, 1]
    thresh = pr_ref[0, 2]
    gt = gt_ref[0, 0]
    mask = gt_ref[0, 1]
    thresh_map = gt_ref[0, 2]
    thresh_mask = gt_ref[0, 3]

    p = jnp.clip(binary, 1e-12, 1.0 - 1e-12)
    logp = jnp.clip(jnp.log(p), -100.0, None)
    log1mp = jnp.clip(jnp.log(1.0 - p), -100.0, None)
    loss = -(gt * logp + (1.0 - gt) * log1mp)

    pos_m = gt * mask
    neg_m = (1.0 - gt) * mask
    negl = loss * neg_m

    row0 = b * _ROWS_PER_B + j * _RB
    neg_ref[pl.ds(row0, _RB), :] = negl

    def _acc_sum(row, val2d):
        sum_ref[row:row + 1, :] += jnp.sum(val2d, axis=0, keepdims=True)

    _acc_sum(0, jnp.abs(thresh - thresh_map) * thresh_mask)
    _acc_sum(1, thresh_mask)
    _acc_sum(2, pos_m)
    _acc_sum(3, neg_m)
    _acc_sum(4, loss * pos_m)

    mm_ref[0:1, :] = jnp.minimum(mm_ref[0:1, :],
                                 jnp.min(loss, axis=0, keepdims=True))
    mm_ref[1:2, :] = jnp.maximum(mm_ref[1:2, :],
                                 jnp.max(loss, axis=0, keepdims=True))
    mm_ref[2:3, :] = jnp.maximum(mm_ref[2:3, :],
                                 jnp.max(negl, axis=0, keepdims=True))

    ptm = thresh_binary * gt * mask
    ppm = thresh_binary * thresh_binary * mask
    ttm = gt * gt * mask

    def _acc_dice(q, val2d):
        dice_ref[q, pl.ds(b, 1), :] += jnp.sum(val2d, axis=0, keepdims=True)

    _acc_dice(0, ptm)
    _acc_dice(1, ptm * loss)
    _acc_dice(2, ppm)
    _acc_dice(3, ppm * loss)
    _acc_dice(4, ttm)
    _acc_dice(5, ttm * loss)

    @pl.when(jnp.logical_and(b == _B - 1, j == _JB - 1))
    def _finalize():
        l1_num = jnp.sum(sum_ref[0, :])
        l1_den = jnp.sum(sum_ref[1, :])
        pos_cnt = jnp.sum(sum_ref[2, :])
        neg_cnt_raw = jnp.sum(sum_ref[3, :])
        pos_loss_sum = jnp.sum(sum_ref[4, :])
        dmin = jnp.min(mm_ref[0, :])
        dmax = jnp.max(mm_ref[1, :])
        neg_max = jnp.max(mm_ref[2, :])

        k_f = jnp.minimum(neg_cnt_raw, pos_cnt * _NEG_RATIO)
        k_i = k_f.astype(jnp.int32)
        k_if = k_i.astype(jnp.float32)

        def count_gt(t):
            def chunk(i, acc):
                blk = neg_ref[pl.ds(i * _CH, _CH), :]
                return acc + jnp.sum((blk > t).astype(jnp.float32))
            return lax.fori_loop(0, _NCH, chunk, jnp.float32(0.0))

        def bisect(_, carry):
            lo, hi = carry
            mid = 0.5 * (lo + hi)
            c = count_gt(mid)
            pred = c > k_if
            return (jnp.where(pred, mid, lo), jnp.where(pred, hi, mid))

        hi0 = neg_max * 1.0000002 + 1e-20
        lo, hi = lax.fori_loop(0, _BISECT_ITERS, bisect,
                               (jnp.float32(0.0), hi0))
        t = hi

        def final_chunk(i, carry):
            c, s = carry
            blk = neg_ref[pl.ds(i * _CH, _CH), :]
            m = blk > t
            c = c + jnp.sum(m.astype(jnp.float32))
            s = s + jnp.sum(jnp.where(m, blk, 0.0))
            return (c, s)

        cnt, ssum = lax.fori_loop(0, _NCH, final_chunk,
                                  (jnp.float32(0.0), jnp.float32(0.0)))
        topk_sum = ssum + (k_if - cnt) * t

        balanced = (pos_loss_sum + topk_sum) / (pos_cnt + k_f + _EPS)
        balanced = balanced * _BAL_SCALE

        a = 1.0 / (dmax - dmin)
        c0 = 1.0 - dmin * a
        dice_total = jnp.float32(0.0)
        for bb in range(_B):
            s_ptm = jnp.sum(dice_ref[0, bb, :])
            s_ptml = jnp.sum(dice_ref[1, bb, :])
            s_ppm = jnp.sum(dice_ref[2, bb, :])
            s_ppml = jnp.sum(dice_ref[3, bb, :])
            s_ttm = jnp.sum(dice_ref[4, bb, :])
            s_ttml = jnp.sum(dice_ref[5, bb, :])
            inter = a * s_ptml + c0 * s_ptm
            union = a * (s_ppml + s_ttml) + c0 * (s_ppm + s_ttm) + 2.0 * _DICE_EPS
            dice_total += 1.0 - 2.0 * inter / union
        dice = dice_total / _B

        l1 = jnp.where(l1_den > 0, l1_num / l1_den, jnp.float32(0.0)) * _L1_SCALE

        out_ref[0, 0] = l1 + balanced + dice


def _dbloss(y_pr4, y_gt4):
    return pl.pallas_call(
        _body,
        grid=(_B, _JB),
        in_specs=[
            pl.BlockSpec((1, 3, _RB, _LANES), lambda b, j: (b, 0, j, 0)),
            pl.BlockSpec((1, 4, _RB, _LANES), lambda b, j: (b, 0, j, 0)),
        ],
        out_specs=pl.BlockSpec(memory_space=pltpu.SMEM),
        out_shape=jax.ShapeDtypeStruct((1, 1), jnp.float32),
        scratch_shapes=[
            pltpu.VMEM((_ROWS, _LANES), jnp.float32),
            pltpu.VMEM((8, _LANES), jnp.float32),
            pltpu.VMEM((8, _LANES), jnp.float32),
            pltpu.VMEM((8, _B, _LANES), jnp.float32),
        ],
        compiler_params=pltpu.CompilerParams(
            dimension_semantics=("arbitrary", "arbitrary"),
        ),
    )(y_pr4, y_gt4)


def kernel(y_pr, y_gt):
    y_pr4 = y_pr.reshape(_B, 3, _ROWS_PER_B, _LANES)
    y_gt4 = y_gt.reshape(_B, 4, _ROWS_PER_B, _LANES)
    out = _dbloss(y_pr4, y_gt4)
    return out[0, 0]


# speedup vs baseline: 17.7946x; 2.4352x over previous
import jax
import jax.numpy as jnp
from jax import lax
from jax.experimental import pallas as pl
from jax.experimental.pallas import tpu as pltpu

_NEG_RATIO = 3.0
_BAL_SCALE = 5.0
_EPS = 1e-09
_L1_SCALE = 10.0
_DICE_EPS = 0.001

_B = 8
_ROWS_PER_B = 400
_LANES = 1024
_RB = 80
_JB = _ROWS_PER_B // _RB
_ROWS = _B * _ROWS_PER_B
_BISECT_ITERS = 30
_CH = 16
_NCH = _ROWS // _CH

_BIG = 3.0e38


def _body(pr_ref, gt_ref, out_ref, neg_ref, sum_ref, mm_ref, dice_ref):
    b = pl.program_id(0)
    j = pl.program_id(1)

    @pl.when(jnp.logical_and(b == 0, j == 0))
    def _init():
        sum_ref[...] = jnp.zeros((8, _LANES), jnp.float32)
        dice_ref[...] = jnp.zeros((8, _B, _LANES), jnp.float32)
        mm_ref[...] = jnp.zeros((8, _LANES), jnp.float32)
        mm_ref[0:1, :] = jnp.full((1, _LANES), _BIG, jnp.float32)
        mm_ref[1:2, :] = jnp.full((1, _LANES), -_BIG, jnp.float32)

    binary = pr_ref[0, 0]
    thresh_binary = pr_ref[0, 1]
    thresh = pr_ref[0, 2]
    gt = gt_ref[0, 0]
    mask = gt_ref[0, 1]
    thresh_map = gt_ref[0, 2]
    thresh_mask = gt_ref[0, 3]

    p = jnp.clip(binary, 1e-12, 1.0 - 1e-12)
    logp = jnp.clip(jnp.log(p), -100.0, None)
    log1mp = jnp.clip(jnp.log(1.0 - p), -100.0, None)
    loss = -(gt * logp + (1.0 - gt) * log1mp)

    pos_m = gt * mask
    neg_m = (1.0 - gt) * mask
    negl = loss * neg_m

    row0 = b * _ROWS_PER_B + j * _RB
    neg_ref[pl.ds(row0, _RB), :] = negl

    def _acc_sum(row, val2d):
        sum_ref[row:row + 1, :] += jnp.sum(val2d, axis=0, keepdims=True)

    _acc_sum(0, jnp.abs(thresh - thresh_map) * thresh_mask)
    _acc_sum(1, thresh_mask)
    _acc_sum(2, pos_m)
    _acc_sum(3, neg_m)
    _acc_sum(4, loss * pos_m)

    mm_ref[0:1, :] = jnp.minimum(mm_ref[0:1, :],
                                 jnp.min(loss, axis=0, keepdims=True))
    mm_ref[1:2, :] = jnp.maximum(mm_ref[1:2, :],
                                 jnp.max(loss, axis=0, keepdims=True))
    mm_ref[2:3, :] = jnp.maximum(mm_ref[2:3, :],
                                 jnp.max(negl, axis=0, keepdims=True))

    ptm = thresh_binary * gt * mask
    ppm = thresh_binary * thresh_binary * mask
    ttm = gt * gt * mask

    def _acc_dice(q, val2d):
        dice_ref[q, pl.ds(b, 1), :] += jnp.sum(val2d, axis=0, keepdims=True)

    _acc_dice(0, ptm)
    _acc_dice(1, ptm * loss)
    _acc_dice(2, ppm)
    _acc_dice(3, ppm * loss)
    _acc_dice(4, ttm)
    _acc_dice(5, ttm * loss)

    @pl.when(jnp.logical_and(b == _B - 1, j == _JB - 1))
    def _finalize():
        l1_num = jnp.sum(sum_ref[0, :])
        l1_den = jnp.sum(sum_ref[1, :])
        pos_cnt = jnp.sum(sum_ref[2, :])
        neg_cnt_raw = jnp.sum(sum_ref[3, :])
        pos_loss_sum = jnp.sum(sum_ref[4, :])
        dmin = jnp.min(mm_ref[0, :])
        dmax = jnp.max(mm_ref[1, :])
        neg_max = jnp.max(mm_ref[2, :])

        k_f = jnp.minimum(neg_cnt_raw, pos_cnt * _NEG_RATIO)
        k_i = k_f.astype(jnp.int32)
        k_if = k_i.astype(jnp.float32)

        def count_gt(t):
            def chunk(i, acc):
                blk = neg_ref[pl.ds(i * _CH, _CH), :]
                return acc + (blk > t).astype(jnp.float32)
            acc = lax.fori_loop(0, _NCH, chunk,
                                jnp.zeros((_CH, _LANES), jnp.float32))
            return jnp.sum(acc)

        def bisect(_, carry):
            lo, hi = carry
            mid = 0.5 * (lo + hi)
            c = count_gt(mid)
            pred = c > k_if
            return (jnp.where(pred, mid, lo), jnp.where(pred, hi, mid))

        hi0 = neg_max * 1.0000002 + 1e-20
        lo, hi = lax.fori_loop(0, _BISECT_ITERS, bisect,
                               (jnp.float32(0.0), hi0))
        t = hi

        def final_chunk(i, carry):
            c, s = carry
            blk = neg_ref[pl.ds(i * _CH, _CH), :]
            m = blk > t
            c = c + m.astype(jnp.float32)
            s = s + jnp.where(m, blk, 0.0)
            return (c, s)

        zed = jnp.zeros((_CH, _LANES), jnp.float32)
        cnt_v, ssum_v = lax.fori_loop(0, _NCH, final_chunk, (zed, zed))
        cnt = jnp.sum(cnt_v)
        ssum = jnp.sum(ssum_v)
        topk_sum = ssum + (k_if - cnt) * t

        balanced = (pos_loss_sum + topk_sum) / (pos_cnt + k_f + _EPS)
        balanced = balanced * _BAL_SCALE

        a = 1.0 / (dmax - dmin)
        c0 = 1.0 - dmin * a
        dice_total = jnp.float32(0.0)
        for bb in range(_B):
            s_ptm = jnp.sum(dice_ref[0, bb, :])
            s_ptml = jnp.sum(dice_ref[1, bb, :])
            s_ppm = jnp.sum(dice_ref[2, bb, :])
            s_ppml = jnp.sum(dice_ref[3, bb, :])
            s_ttm = jnp.sum(dice_ref[4, bb, :])
            s_ttml = jnp.sum(dice_ref[5, bb, :])
            inter = a * s_ptml + c0 * s_ptm
            union = a * (s_ppml + s_ttml) + c0 * (s_ppm + s_ttm) + 2.0 * _DICE_EPS
            dice_total += 1.0 - 2.0 * inter / union
        dice = dice_total / _B

        l1 = jnp.where(l1_den > 0, l1_num / l1_den, jnp.float32(0.0)) * _L1_SCALE

        out_ref[0, 0] = l1 + balanced + dice


def _dbloss(y_pr4, y_gt4):
    return pl.pallas_call(
        _body,
        grid=(_B, _JB),
        in_specs=[
            pl.BlockSpec((1, 3, _RB, _LANES), lambda b, j: (b, 0, j, 0)),
            pl.BlockSpec((1, 4, _RB, _LANES), lambda b, j: (b, 0, j, 0)),
        ],
        out_specs=pl.BlockSpec(memory_space=pltpu.SMEM),
        out_shape=jax.ShapeDtypeStruct((1, 1), jnp.float32),
        scratch_shapes=[
            pltpu.VMEM((_ROWS, _LANES), jnp.float32),
            pltpu.VMEM((8, _LANES), jnp.float32),
            pltpu.VMEM((8, _LANES), jnp.float32),
            pltpu.VMEM((8, _B, _LANES), jnp.float32),
        ],
        compiler_params=pltpu.CompilerParams(
            dimension_semantics=("arbitrary", "arbitrary"),
        ),
    )(y_pr4, y_gt4)


def kernel(y_pr, y_gt):
    y_pr4 = y_pr.reshape(_B, 3, _ROWS_PER_B, _LANES)
    y_gt4 = y_gt.reshape(_B, 4, _ROWS_PER_B, _LANES)
    out = _dbloss(y_pr4, y_gt4)
    return out[0, 0]


# speedup vs baseline: 20.5850x; 1.1568x over previous
import jax
import jax.numpy as jnp
from jax import lax
from jax.experimental import pallas as pl
from jax.experimental.pallas import tpu as pltpu

_NEG_RATIO = 3.0
_BAL_SCALE = 5.0
_EPS = 1e-09
_L1_SCALE = 10.0
_DICE_EPS = 0.001

_B = 8
_ROWS_PER_B = 400
_LANES = 1024
_RB = 80
_JB = _ROWS_PER_B // _RB
_ROWS = _B * _ROWS_PER_B
_CH = 16
_NCH = _ROWS // _CH
_SUB_ROWS = 256
_SUB_ITERS = 18
_FULL_ITERS = 14
_BRACKET = 0.04

_BIG = 3.0e38


def _body(pr_ref, gt_ref, out_ref, neg_ref, sum_ref, mm_ref, dice_ref):
    b = pl.program_id(0)
    j = pl.program_id(1)

    @pl.when(jnp.logical_and(b == 0, j == 0))
    def _init():
        sum_ref[...] = jnp.zeros((8, _LANES), jnp.float32)
        dice_ref[...] = jnp.zeros((8, _B, _LANES), jnp.float32)
        mm_ref[...] = jnp.zeros((8, _LANES), jnp.float32)
        mm_ref[0:1, :] = jnp.full((1, _LANES), _BIG, jnp.float32)
        mm_ref[1:2, :] = jnp.full((1, _LANES), -_BIG, jnp.float32)

    binary = pr_ref[0, 0]
    thresh_binary = pr_ref[0, 1]
    thresh = pr_ref[0, 2]
    gt = gt_ref[0, 0]
    mask = gt_ref[0, 1]
    thresh_map = gt_ref[0, 2]
    thresh_mask = gt_ref[0, 3]

    p = jnp.clip(binary, 1e-12, 1.0 - 1e-12)
    logp = jnp.clip(jnp.log(p), -100.0, None)
    log1mp = jnp.clip(jnp.log(1.0 - p), -100.0, None)
    loss = -(gt * logp + (1.0 - gt) * log1mp)

    pos_m = gt * mask
    neg_m = (1.0 - gt) * mask
    negl = loss * neg_m

    row0 = b * _ROWS_PER_B + j * _RB
    neg_ref[pl.ds(row0, _RB), :] = negl

    def _acc_sum(row, val2d):
        sum_ref[row:row + 1, :] += jnp.sum(val2d, axis=0, keepdims=True)

    _acc_sum(0, jnp.abs(thresh - thresh_map) * thresh_mask)
    _acc_sum(1, thresh_mask)
    _acc_sum(2, pos_m)
    _acc_sum(3, neg_m)
    _acc_sum(4, loss * pos_m)

    mm_ref[0:1, :] = jnp.minimum(mm_ref[0:1, :],
                                 jnp.min(loss, axis=0, keepdims=True))
    mm_ref[1:2, :] = jnp.maximum(mm_ref[1:2, :],
                                 jnp.max(loss, axis=0, keepdims=True))
    mm_ref[2:3, :] = jnp.maximum(mm_ref[2:3, :],
                                 jnp.max(negl, axis=0, keepdims=True))

    ptm = thresh_binary * gt * mask
    ppm = thresh_binary * thresh_binary * mask
    ttm = gt * gt * mask

    def _acc_dice(q, val2d):
        dice_ref[q, pl.ds(b, 1), :] += jnp.sum(val2d, axis=0, keepdims=True)

    _acc_dice(0, ptm)
    _acc_dice(1, ptm * loss)
    _acc_dice(2, ppm)
    _acc_dice(3, ppm * loss)
    _acc_dice(4, ttm)
    _acc_dice(5, ttm * loss)

    @pl.when(jnp.logical_and(b == _B - 1, j == _JB - 1))
    def _finalize():
        l1_num = jnp.sum(sum_ref[0, :])
        l1_den = jnp.sum(sum_ref[1, :])
        pos_cnt = jnp.sum(sum_ref[2, :])
        neg_cnt_raw = jnp.sum(sum_ref[3, :])
        pos_loss_sum = jnp.sum(sum_ref[4, :])
        dmin = jnp.min(mm_ref[0, :])
        dmax = jnp.max(mm_ref[1, :])
        neg_max = jnp.max(mm_ref[2, :])

        k_f = jnp.minimum(neg_cnt_raw, pos_cnt * _NEG_RATIO)
        k_i = k_f.astype(jnp.int32)
        k_if = k_i.astype(jnp.float32)

        zed = jnp.zeros((_CH, _LANES), jnp.float32)

        def count_gt(t, nch):
            def chunk(i, acc):
                blk = neg_ref[pl.ds(i * _CH, _CH), :]
                return acc + (blk > t).astype(jnp.float32)
            return jnp.sum(lax.fori_loop(0, nch, chunk, zed))

        def count2_gt(t1, t2):
            def chunk(i, carry):
                a1, a2 = carry
                blk = neg_ref[pl.ds(i * _CH, _CH), :]
                return (a1 + (blk > t1).astype(jnp.float32),
                        a2 + (blk > t2).astype(jnp.float32))
            a1, a2 = lax.fori_loop(0, _NCH, chunk, (zed, zed))
            return jnp.sum(a1), jnp.sum(a2)

        hi0 = neg_max * 1.0000002 + 1e-20

        k_sub = k_if * (_SUB_ROWS / _ROWS)

        def sub_bisect(_, carry):
            lo, hi = carry
            mid = 0.5 * (lo + hi)
            c = count_gt(mid, _SUB_ROWS // _CH)
            pred = c > k_sub
            return (jnp.where(pred, mid, lo), jnp.where(pred, hi, mid))

        _, t_est = lax.fori_loop(0, _SUB_ITERS, sub_bisect,
                                 (jnp.float32(0.0), hi0))

        lo0 = jnp.maximum(t_est - _BRACKET, 0.0)
        hi1 = jnp.minimum(t_est + _BRACKET, hi0)
        c_lo, c_hi = count2_gt(lo0, hi1)
        lo = jnp.where(c_lo > k_if, lo0, jnp.float32(0.0))
        hi = jnp.where(c_hi <= k_if, hi1, hi0)

        def bisect(_, carry):
            lo, hi = carry
            mid = 0.5 * (lo + hi)
            c = count_gt(mid, _NCH)
            pred = c > k_if
            return (jnp.where(pred, mid, lo), jnp.where(pred, hi, mid))

        lo, hi = lax.fori_loop(0, _FULL_ITERS, bisect, (lo, hi))
        t = hi

        def final_chunk(i, carry):
            c, s = carry
            blk = neg_ref[pl.ds(i * _CH, _CH), :]
            m = blk > t
            c = c + m.astype(jnp.float32)
            s = s + jnp.where(m, blk, 0.0)
            return (c, s)

        cnt_v, ssum_v = lax.fori_loop(0, _NCH, final_chunk, (zed, zed))
        cnt = jnp.sum(cnt_v)
        ssum = jnp.sum(ssum_v)
        topk_sum = ssum + (k_if - cnt) * t

        balanced = (pos_loss_sum + topk_sum) / (pos_cnt + k_f + _EPS)
        balanced = balanced * _BAL_SCALE

        a = 1.0 / (dmax - dmin)
        c0 = 1.0 - dmin * a
        dice_total = jnp.float32(0.0)
        for bb in range(_B):
            s_ptm = jnp.sum(dice_ref[0, bb, :])
            s_ptml = jnp.sum(dice_ref[1, bb, :])
            s_ppm = jnp.sum(dice_ref[2, bb, :])
            s_ppml = jnp.sum(dice_ref[3, bb, :])
            s_ttm = jnp.sum(dice_ref[4, bb, :])
            s_ttml = jnp.sum(dice_ref[5, bb, :])
            inter = a * s_ptml + c0 * s_ptm
            union = a * (s_ppml + s_ttml) + c0 * (s_ppm + s_ttm) + 2.0 * _DICE_EPS
            dice_total += 1.0 - 2.0 * inter / union
        dice = dice_total / _B

        l1 = jnp.where(l1_den > 0, l1_num / l1_den, jnp.float32(0.0)) * _L1_SCALE

        out_ref[0, 0] = l1 + balanced + dice


def _dbloss(y_pr4, y_gt4):
    return pl.pallas_call(
        _body,
        grid=(_B, _JB),
        in_specs=[
            pl.BlockSpec((1, 3, _RB, _LANES), lambda b, j: (b, 0, j, 0)),
            pl.BlockSpec((1, 4, _RB, _LANES), lambda b, j: (b, 0, j, 0)),
        ],
        out_specs=pl.BlockSpec(memory_space=pltpu.SMEM),
        out_shape=jax.ShapeDtypeStruct((1, 1), jnp.float32),
        scratch_shapes=[
            pltpu.VMEM((_ROWS, _LANES), jnp.float32),
            pltpu.VMEM((8, _LANES), jnp.float32),
            pltpu.VMEM((8, _LANES), jnp.float32),
            pltpu.VMEM((8, _B, _LANES), jnp.float32),
        ],
        compiler_params=pltpu.CompilerParams(
            dimension_semantics=("arbitrary", "arbitrary"),
        ),
    )(y_pr4, y_gt4)


def kernel(y_pr, y_gt):
    y_pr4 = y_pr.reshape(_B, 3, _ROWS_PER_B, _LANES)
    y_gt4 = y_gt.reshape(_B, 4, _ROWS_PER_B, _LANES)
    out = _dbloss(y_pr4, y_gt4)
    return out[0, 0]
